# initial kernel scaffold (unmeasured)
import jax
import jax.numpy as jnp
from jax import lax
from jax.experimental import pallas as pl
from jax.experimental.pallas import tpu as pltpu


def kernel(
    x,
):
    def body(*refs):
        pass

    out_shape = jax.ShapeDtypeStruct(..., jnp.float32)
    return pl.pallas_call(body, out_shape=out_shape)(...)


# baseline (device time: 31244 ns/iter reference)
import jax
import jax.numpy as jnp
from jax import lax
from jax.experimental import pallas as pl
from jax.experimental.pallas import tpu as pltpu


def kernel(x):
    _, m, n_shard = x.shape
    n_full = 2 * n_shard

    def body(x_ref, out_ref, comm_ref, send_sems, recv_sems):
        my_x = lax.axis_index("x")
        my_y = lax.axis_index("y")

        barrier_sem = pltpu.get_barrier_semaphore()
        for nbr in [(1 - my_x, my_y), (my_x, 1 - my_y)]:
            pl.semaphore_signal(
                barrier_sem, inc=1,
                device_id=nbr, device_id_type=pl.DeviceIdType.MESH,
            )
        pl.semaphore_wait(barrier_sem, 2)

        rdma_x = pltpu.make_async_remote_copy(
            src_ref=x_ref.at[0],
            dst_ref=comm_ref.at[0],
            send_sem=send_sems.at[0],
            recv_sem=recv_sems.at[0],
            device_id=(1 - my_x, my_y),
            device_id_type=pl.DeviceIdType.MESH,
        )
        rdma_x.start()
        rdma_x.wait()

        comm_ref[1, :, :] = x_ref[0, :, :] + comm_ref[0, :, :]

        rdma_y = pltpu.make_async_remote_copy(
            src_ref=comm_ref.at[1],
            dst_ref=comm_ref.at[2],
            send_sem=send_sems.at[1],
            recv_sem=recv_sems.at[1],
            device_id=(my_x, 1 - my_y),
            device_id_type=pl.DeviceIdType.MESH,
        )
        rdma_y.start()
        out_ref[:, pl.ds(my_y * n_shard, n_shard)] = comm_ref[1, :, :]
        rdma_y.wait()
        out_ref[:, pl.ds((1 - my_y) * n_shard, n_shard)] = comm_ref[2, :, :]

    return pl.pallas_call(
        body,
        out_shape=jax.ShapeDtypeStruct((m, n_full), x.dtype),
        in_specs=[pl.BlockSpec(memory_space=pltpu.VMEM)],
        out_specs=pl.BlockSpec(memory_space=pltpu.VMEM),
        scratch_shapes=[
            pltpu.VMEM((3, m, n_shard), x.dtype),
            pltpu.SemaphoreType.DMA((2,)),
            pltpu.SemaphoreType.DMA((2,)),
        ],
        compiler_params=pltpu.CompilerParams(collective_id=0),
    )(x)


# device time: 22832 ns/iter; 1.3684x vs baseline; 1.3684x over previous
import jax
import jax.numpy as jnp
from jax import lax
from jax.experimental import pallas as pl
from jax.experimental.pallas import tpu as pltpu

C = 4


def kernel(x):
    _, m, n_shard = x.shape
    n_full = 2 * n_shard
    mc = m // C

    def body(x_ref, out_ref, comm_ref, xs_sems, xr_sems, ys_sems, yr_sems):
        my_x = lax.axis_index("x")
        my_y = lax.axis_index("y")
        x_peer = (1 - my_x, my_y)
        y_peer = (my_x, 1 - my_y)
        my_col = my_y * n_shard

        barrier_sem = pltpu.get_barrier_semaphore()
        for nbr in [x_peer, y_peer]:
            pl.semaphore_signal(
                barrier_sem, inc=1,
                device_id=nbr, device_id_type=pl.DeviceIdType.MESH,
            )
        pl.semaphore_wait(barrier_sem, 2)

        rdma_x = []
        for c in range(C):
            r = pltpu.make_async_remote_copy(
                src_ref=x_ref.at[0, pl.ds(c * mc, mc), :],
                dst_ref=comm_ref.at[pl.ds(c * mc, mc), :],
                send_sem=xs_sems.at[c],
                recv_sem=xr_sems.at[c],
                device_id=x_peer,
                device_id_type=pl.DeviceIdType.MESH,
            )
            r.start()
            rdma_x.append(r)

        rdma_y = []
        for c in range(C):
            rdma_x[c].wait_recv()
            rows = pl.ds(c * mc, mc)
            out_ref[rows, pl.ds(my_col, n_shard)] = (
                x_ref[0, rows, :] + comm_ref[rows, :]
            )
            r = pltpu.make_async_remote_copy(
                src_ref=out_ref.at[rows, pl.ds(my_col, n_shard)],
                dst_ref=out_ref.at[rows, pl.ds(my_col, n_shard)],
                send_sem=ys_sems.at[c],
                recv_sem=yr_sems.at[c],
                device_id=y_peer,
                device_id_type=pl.DeviceIdType.MESH,
            )
            r.start()
            rdma_y.append(r)

        for c in range(C):
            rdma_y[c].wait_recv()
        for c in range(C):
            rdma_x[c].wait_send()
            rdma_y[c].wait_send()

    return pl.pallas_call(
        body,
        out_shape=jax.ShapeDtypeStruct((m, n_full), x.dtype),
        in_specs=[pl.BlockSpec(memory_space=pltpu.VMEM)],
        out_specs=pl.BlockSpec(memory_space=pltpu.VMEM),
        scratch_shapes=[
            pltpu.VMEM((m, n_shard), x.dtype),
            pltpu.SemaphoreType.DMA((C,)),
            pltpu.SemaphoreType.DMA((C,)),
            pltpu.SemaphoreType.DMA((C,)),
            pltpu.SemaphoreType.DMA((C,)),
        ],
        compiler_params=pltpu.CompilerParams(collective_id=0),
    )(x)


# device time: 21470 ns/iter; 1.4552x vs baseline; 1.0634x over previous
import jax
import jax.numpy as jnp
from jax import lax
from jax.experimental import pallas as pl
from jax.experimental.pallas import tpu as pltpu

C = 8


def kernel(x):
    _, m, n_shard = x.shape
    n_full = 2 * n_shard
    mc = m // C

    def body(x_ref, out_ref, comm_ref, xs_sems, xr_sems, ys_sems, yr_sems):
        my_x = lax.axis_index("x")
        my_y = lax.axis_index("y")
        x_peer = (1 - my_x, my_y)
        y_peer = (my_x, 1 - my_y)
        my_col = my_y * n_shard

        barrier_sem = pltpu.get_barrier_semaphore()
        for nbr in [x_peer, y_peer]:
            pl.semaphore_signal(
                barrier_sem, inc=1,
                device_id=nbr, device_id_type=pl.DeviceIdType.MESH,
            )
        pl.semaphore_wait(barrier_sem, 2)

        rdma_x = []
        for c in range(C):
            r = pltpu.make_async_remote_copy(
                src_ref=x_ref.at[0, pl.ds(c * mc, mc), :],
                dst_ref=comm_ref.at[pl.ds(c * mc, mc), :],
                send_sem=xs_sems.at[c],
                recv_sem=xr_sems.at[c],
                device_id=x_peer,
                device_id_type=pl.DeviceIdType.MESH,
            )
            r.start()
            rdma_x.append(r)

        rdma_y = []
        for c in range(C):
            rdma_x[c].wait_recv()
            rows = pl.ds(c * mc, mc)
            out_ref[rows, pl.ds(my_col, n_shard)] = (
                x_ref[0, rows, :] + comm_ref[rows, :]
            )
            r = pltpu.make_async_remote_copy(
                src_ref=out_ref.at[rows, pl.ds(my_col, n_shard)],
                dst_ref=out_ref.at[rows, pl.ds(my_col, n_shard)],
                send_sem=ys_sems.at[c],
                recv_sem=yr_sems.at[c],
                device_id=y_peer,
                device_id_type=pl.DeviceIdType.MESH,
            )
            r.start()
            rdma_y.append(r)

        for c in range(C):
            rdma_y[c].wait_recv()
        for c in range(C):
            rdma_x[c].wait_send()
            rdma_y[c].wait_send()

    return pl.pallas_call(
        body,
        out_shape=jax.ShapeDtypeStruct((m, n_full), x.dtype),
        in_specs=[pl.BlockSpec(memory_space=pltpu.VMEM)],
        out_specs=pl.BlockSpec(memory_space=pltpu.VMEM),
        scratch_shapes=[
            pltpu.VMEM((m, n_shard), x.dtype),
            pltpu.SemaphoreType.DMA((C,)),
            pltpu.SemaphoreType.DMA((C,)),
            pltpu.SemaphoreType.DMA((C,)),
            pltpu.SemaphoreType.DMA((C,)),
        ],
        compiler_params=pltpu.CompilerParams(collective_id=0),
    )(x)


# device time: 21052 ns/iter; 1.4841x vs baseline; 1.0199x over previous
import jax
import jax.numpy as jnp
from jax import lax
from jax.experimental import pallas as pl
from jax.experimental.pallas import tpu as pltpu

C = 16


def kernel(x):
    _, m, n_shard = x.shape
    n_full = 2 * n_shard
    mc = m // C

    def body(x_ref, out_ref, comm_ref, xs_sems, xr_sems, ys_sems, yr_sems):
        my_x = lax.axis_index("x")
        my_y = lax.axis_index("y")
        x_peer = (1 - my_x, my_y)
        y_peer = (my_x, 1 - my_y)
        my_col = my_y * n_shard

        barrier_sem = pltpu.get_barrier_semaphore()
        for nbr in [x_peer, y_peer]:
            pl.semaphore_signal(
                barrier_sem, inc=1,
                device_id=nbr, device_id_type=pl.DeviceIdType.MESH,
            )
        pl.semaphore_wait(barrier_sem, 2)

        rdma_x = []
        for c in range(C):
            r = pltpu.make_async_remote_copy(
                src_ref=x_ref.at[0, pl.ds(c * mc, mc), :],
                dst_ref=comm_ref.at[pl.ds(c * mc, mc), :],
                send_sem=xs_sems.at[c],
                recv_sem=xr_sems.at[c],
                device_id=x_peer,
                device_id_type=pl.DeviceIdType.MESH,
            )
            r.start()
            rdma_x.append(r)

        rdma_y = []
        for c in range(C):
            rdma_x[c].wait_recv()
            rows = pl.ds(c * mc, mc)
            out_ref[rows, pl.ds(my_col, n_shard)] = (
                x_ref[0, rows, :] + comm_ref[rows, :]
            )
            r = pltpu.make_async_remote_copy(
                src_ref=out_ref.at[rows, pl.ds(my_col, n_shard)],
                dst_ref=out_ref.at[rows, pl.ds(my_col, n_shard)],
                send_sem=ys_sems.at[c],
                recv_sem=yr_sems.at[c],
                device_id=y_peer,
                device_id_type=pl.DeviceIdType.MESH,
            )
            r.start()
            rdma_y.append(r)

        for c in range(C):
            rdma_y[c].wait_recv()
        for c in range(C):
            rdma_x[c].wait_send()
            rdma_y[c].wait_send()

    return pl.pallas_call(
        body,
        out_shape=jax.ShapeDtypeStruct((m, n_full), x.dtype),
        in_specs=[pl.BlockSpec(memory_space=pltpu.VMEM)],
        out_specs=pl.BlockSpec(memory_space=pltpu.VMEM),
        scratch_shapes=[
            pltpu.VMEM((m, n_shard), x.dtype),
            pltpu.SemaphoreType.DMA((C,)),
            pltpu.SemaphoreType.DMA((C,)),
            pltpu.SemaphoreType.DMA((C,)),
            pltpu.SemaphoreType.DMA((C,)),
        ],
        compiler_params=pltpu.CompilerParams(collective_id=0),
    )(x)


# device time: 21010 ns/iter; 1.4871x vs baseline; 1.0020x over previous
import jax
import jax.numpy as jnp
from jax import lax
from jax.experimental import pallas as pl
from jax.experimental.pallas import tpu as pltpu

C = 16


def kernel(x):
    _, m, n_shard = x.shape
    n_full = 2 * n_shard
    mc = m // C

    def body(x_ref, out_ref, comm_ref,
             xs_sems, xr_sems, ys_sems, yr_sems, cp_sems):
        my_x = lax.axis_index("x")
        my_y = lax.axis_index("y")
        x_peer = (1 - my_x, my_y)
        y_peer = (my_x, 1 - my_y)
        my_col = my_y * n_shard

        barrier_sem = pltpu.get_barrier_semaphore()
        for nbr in [x_peer, y_peer]:
            pl.semaphore_signal(
                barrier_sem, inc=1,
                device_id=nbr, device_id_type=pl.DeviceIdType.MESH,
            )
        pl.semaphore_wait(barrier_sem, 2)

        rdma_x = []
        for c in range(C):
            r = pltpu.make_async_remote_copy(
                src_ref=x_ref.at[0, pl.ds(c * mc, mc), :],
                dst_ref=comm_ref.at[pl.ds(c * mc, mc), :],
                send_sem=xs_sems.at[c],
                recv_sem=xr_sems.at[c],
                device_id=x_peer,
                device_id_type=pl.DeviceIdType.MESH,
            )
            r.start()
            rdma_x.append(r)

        rdma_y = []
        copies = []
        for c in range(C):
            rdma_x[c].wait_recv()
            rows = pl.ds(c * mc, mc)
            comm_ref[rows, :] = comm_ref[rows, :] + x_ref[0, rows, :]
            r = pltpu.make_async_remote_copy(
                src_ref=comm_ref.at[rows, :],
                dst_ref=out_ref.at[rows, pl.ds(my_col, n_shard)],
                send_sem=ys_sems.at[c],
                recv_sem=yr_sems.at[c],
                device_id=y_peer,
                device_id_type=pl.DeviceIdType.MESH,
            )
            r.start()
            rdma_y.append(r)
            cp = pltpu.make_async_copy(
                comm_ref.at[rows, :],
                out_ref.at[rows, pl.ds(my_col, n_shard)],
                cp_sems.at[c],
            )
            cp.start()
            copies.append(cp)

        for c in range(C):
            rdma_y[c].wait_recv()
            copies[c].wait()
        for c in range(C):
            rdma_x[c].wait_send()
            rdma_y[c].wait_send()

    return pl.pallas_call(
        body,
        out_shape=jax.ShapeDtypeStruct((m, n_full), x.dtype),
        in_specs=[pl.BlockSpec(memory_space=pltpu.VMEM)],
        out_specs=pl.BlockSpec(memory_space=pltpu.MemorySpace.HBM),
        scratch_shapes=[
            pltpu.VMEM((m, n_shard), x.dtype),
            pltpu.SemaphoreType.DMA((C,)),
            pltpu.SemaphoreType.DMA((C,)),
            pltpu.SemaphoreType.DMA((C,)),
            pltpu.SemaphoreType.DMA((C,)),
            pltpu.SemaphoreType.DMA((C,)),
        ],
        compiler_params=pltpu.CompilerParams(collective_id=0),
    )(x)
